# baseline (speedup 1.0000x reference)
import functools

import jax
import jax.numpy as jnp
from jax.experimental import pallas as pl
from jax.experimental.pallas import tpu as pltpu

_TILE = 512


def _router_kernel(x_ref, w_ref, o_ref):
    logits = jax.lax.dot_general(
        x_ref[...], w_ref[...],
        dimension_numbers=(((1,), (1,)), ((), ())),
        preferred_element_type=jnp.float32,
    )
    m = jnp.max(logits, axis=-1, keepdims=True)
    e = jnp.exp(logits - m)
    o_ref[...] = e / jnp.sum(e, axis=-1, keepdims=True)


@functools.partial(jax.jit, static_argnames=("interpret",))
def kernel(x, expert_embeddings, interpret=False):
    B, S, H = x.shape
    E = expert_embeddings.shape[0]
    rows = B * S
    x2 = x.reshape(rows, H)
    grid = (rows // _TILE,)
    out = pl.pallas_call(
        _router_kernel,
        grid=grid,
        in_specs=[
            pl.BlockSpec((_TILE, H), lambda i: (i, 0)),
            pl.BlockSpec((E, H), lambda i: (0, 0)),
        ],
        out_specs=pl.BlockSpec((_TILE, E), lambda i: (i, 0)),
        out_shape=jax.ShapeDtypeStruct((rows, E), jnp.float32),
        compiler_params=pltpu.CompilerParams(
            dimension_semantics=("arbitrary",),
        ),
        interpret=interpret,
    )(x2, expert_embeddings)
    return out.reshape(B, S, E)


# speedup vs baseline: 1.0053x; 1.0053x over previous
import functools

import jax
import jax.numpy as jnp
from jax.experimental import pallas as pl
from jax.experimental.pallas import tpu as pltpu

_TILE = 1024


def _router_kernel(x_ref, w_ref, o_ref):
    logits = jax.lax.dot_general(
        x_ref[...], w_ref[...],
        dimension_numbers=(((1,), (1,)), ((), ())),
        preferred_element_type=jnp.float32,
    )
    m = jnp.max(logits, axis=-1, keepdims=True)
    e = jnp.exp(logits - m)
    o_ref[...] = e / jnp.sum(e, axis=-1, keepdims=True)


@functools.partial(jax.jit, static_argnames=("interpret",))
def kernel(x, expert_embeddings, interpret=False):
    B, S, H = x.shape
    E = expert_embeddings.shape[0]
    rows = B * S
    x2 = x.reshape(rows, H)
    grid = (rows // _TILE,)
    out = pl.pallas_call(
        _router_kernel,
        grid=grid,
        in_specs=[
            pl.BlockSpec((_TILE, H), lambda i: (i, 0)),
            pl.BlockSpec((E, H), lambda i: (0, 0)),
        ],
        out_specs=pl.BlockSpec((_TILE, E), lambda i: (i, 0)),
        out_shape=jax.ShapeDtypeStruct((rows, E), jnp.float32),
        compiler_params=pltpu.CompilerParams(
            dimension_semantics=("parallel",),
        ),
        interpret=interpret,
    )(x2, expert_embeddings)
    return out.reshape(B, S, E)
